# Initial kernel scaffold; baseline (speedup 1.0000x reference)
import jax
import jax.numpy as jnp
from jax.experimental import pallas as pl


def kernel(epoch, logits, targets, group_ids, aux_ce_loss, u, c, c_buf):
    raise NotImplementedError("write your pallas kernel here")


# speedup vs baseline: 1.0497x; 1.0497x over previous
import jax
import jax.numpy as jnp
from jax.experimental import pallas as pl
from jax.experimental.pallas import tpu as pltpu

ALPHA = 0.2
GAMMA = 0.2
THETA = 0.1
LAMDA = 0.1
N_GROUPS = 10
N_GPB = 8

ROWS = 16384
CLASSES = 1000
R = 512
G = ROWS // R
BPS = (ROWS // N_GPB) // R
INV_BPG = 1.0 / (ROWS // N_GPB)


def _ce_kernel(gid_ref, u_ref, aux_ref, ccb_ref,
               logits_ref, targets_ref, out_ref, acc_ref, us_ref):
    pid = pl.program_id(0)

    @pl.when(pid == 0)
    def _init():
        for k in range(N_GPB):
            acc_ref[k] = 0.0

    x = logits_ref[...]
    t = targets_ref[0, 0, :]
    m = jnp.max(x, axis=1)
    e = jnp.exp(x - m[:, None])
    s = jnp.sum(e, axis=1)
    lse = jnp.log(s) + m
    col = jax.lax.broadcasted_iota(jnp.int32, x.shape, 1)
    tgt = jnp.sum(jnp.where(col == t[:, None], x, 0.0), axis=1)
    block_sum = jnp.sum(lse - tgt)
    g = pid // BPS
    acc_ref[g] += block_sum

    @pl.when(pid == G - 1)
    def _finish():
        c = ccb_ref[0]
        c_buf = ccb_ref[1]
        for j in range(N_GROUPS):
            us_ref[j] = u_ref[j]
        for k in range(N_GPB):
            ce_d = acc_ref[k] * INV_BPG
            gk = gid_ref[k]
            ug = u_ref[gk]
            val = ug + GAMMA * (ce_d - c - ug) + THETA * (ce_d - c - (aux_ref[k] - c_buf))
            us_ref[gk] = val
        total = 0.0
        for k in range(N_GPB):
            w = jnp.minimum(jnp.exp(us_ref[gid_ref[k]] / LAMDA), 1.0 / ALPHA)
            total = total + w * (acc_ref[k] * INV_BPG)
        out_ref[0] = total / N_GPB


@jax.jit
def _run(logits, targets3, gid, u, aux, ccb):
    return pl.pallas_call(
        _ce_kernel,
        grid=(G,),
        in_specs=[
            pl.BlockSpec(memory_space=pltpu.SMEM),
            pl.BlockSpec(memory_space=pltpu.SMEM),
            pl.BlockSpec(memory_space=pltpu.SMEM),
            pl.BlockSpec(memory_space=pltpu.SMEM),
            pl.BlockSpec((R, CLASSES), lambda i: (i, 0)),
            pl.BlockSpec((1, 1, R), lambda i: (i, 0, 0)),
        ],
        out_specs=pl.BlockSpec(memory_space=pltpu.SMEM),
        out_shape=jax.ShapeDtypeStruct((1,), jnp.float32),
        scratch_shapes=[
            pltpu.SMEM((N_GPB,), jnp.float32),
            pltpu.SMEM((N_GROUPS,), jnp.float32),
        ],
        compiler_params=pltpu.CompilerParams(
            dimension_semantics=("arbitrary",)),
    )(gid, u, aux, ccb, logits, targets3)


def kernel(epoch, logits, targets, group_ids, aux_ce_loss, u, c, c_buf):
    gid = group_ids[:: ROWS // N_GPB]
    targets3 = targets.astype(jnp.int32).reshape(G, 1, R)
    ccb = jnp.stack([jnp.asarray(c, jnp.float32), jnp.asarray(c_buf, jnp.float32)])
    out = _run(logits, targets3, gid, u, aux_ce_loss, ccb)
    return out[0]


# speedup vs baseline: 1.1539x; 1.0993x over previous
import jax
import jax.numpy as jnp
from jax.experimental import pallas as pl
from jax.experimental.pallas import tpu as pltpu

ALPHA = 0.2
GAMMA = 0.2
THETA = 0.1
LAMDA = 0.1
N_GROUPS = 10
N_GPB = 8

ROWS = 16384
CLASSES = 1000
R = 1024
G = ROWS // R
BPS = (ROWS // N_GPB) // R
INV_BPG = 1.0 / (ROWS // N_GPB)


def _ce_kernel(gid_ref, u_ref, aux_ref, ccb_ref,
               logits_ref, targets_ref, out_ref, acc_ref, us_ref):
    pid = pl.program_id(0)

    @pl.when(pid == 0)
    def _init():
        for k in range(N_GPB):
            acc_ref[k] = 0.0

    x = logits_ref[...]
    t = targets_ref[0, 0, :]
    m = jnp.max(x, axis=1)
    e = jnp.exp(x - m[:, None])
    s = jnp.sum(e, axis=1)
    lse = jnp.log(s) + m
    col = jax.lax.broadcasted_iota(jnp.int32, x.shape, 1)
    tgt = jnp.sum(jnp.where(col == t[:, None], x, 0.0), axis=1)
    block_sum = jnp.sum(lse - tgt)
    g = pid // BPS
    acc_ref[g] += block_sum

    @pl.when(pid == G - 1)
    def _finish():
        c = ccb_ref[0]
        c_buf = ccb_ref[1]
        for j in range(N_GROUPS):
            us_ref[j] = u_ref[j]
        for k in range(N_GPB):
            ce_d = acc_ref[k] * INV_BPG
            gk = gid_ref[k]
            ug = u_ref[gk]
            val = ug + GAMMA * (ce_d - c - ug) + THETA * (ce_d - c - (aux_ref[k] - c_buf))
            us_ref[gk] = val
        total = 0.0
        for k in range(N_GPB):
            w = jnp.minimum(jnp.exp(us_ref[gid_ref[k]] / LAMDA), 1.0 / ALPHA)
            total = total + w * (acc_ref[k] * INV_BPG)
        out_ref[0] = total / N_GPB


@jax.jit
def _run(logits, targets3, gid, u, aux, ccb):
    return pl.pallas_call(
        _ce_kernel,
        grid=(G,),
        in_specs=[
            pl.BlockSpec(memory_space=pltpu.SMEM),
            pl.BlockSpec(memory_space=pltpu.SMEM),
            pl.BlockSpec(memory_space=pltpu.SMEM),
            pl.BlockSpec(memory_space=pltpu.SMEM),
            pl.BlockSpec((R, CLASSES), lambda i: (i, 0)),
            pl.BlockSpec((1, 1, R), lambda i: (i, 0, 0)),
        ],
        out_specs=pl.BlockSpec(memory_space=pltpu.SMEM),
        out_shape=jax.ShapeDtypeStruct((1,), jnp.float32),
        scratch_shapes=[
            pltpu.SMEM((N_GPB,), jnp.float32),
            pltpu.SMEM((N_GROUPS,), jnp.float32),
        ],
        compiler_params=pltpu.CompilerParams(
            dimension_semantics=("arbitrary",)),
    )(gid, u, aux, ccb, logits, targets3)


def kernel(epoch, logits, targets, group_ids, aux_ce_loss, u, c, c_buf):
    gid = group_ids[:: ROWS // N_GPB]
    targets3 = targets.astype(jnp.int32).reshape(G, 1, R)
    ccb = jnp.stack([jnp.asarray(c, jnp.float32), jnp.asarray(c_buf, jnp.float32)])
    out = _run(logits, targets3, gid, u, aux_ce_loss, ccb)
    return out[0]


# speedup vs baseline: 1.2015x; 1.0413x over previous
import jax
import jax.numpy as jnp
from jax.experimental import pallas as pl
from jax.experimental.pallas import tpu as pltpu

ALPHA = 0.2
GAMMA = 0.2
THETA = 0.1
LAMDA = 0.1
N_GROUPS = 10
N_GPB = 8

ROWS = 16384
CLASSES = 1000
R = 2048
G = ROWS // R
BPS = (ROWS // N_GPB) // R
INV_BPG = 1.0 / (ROWS // N_GPB)


def _ce_kernel(gid_ref, u_ref, aux_ref, ccb_ref,
               logits_ref, targets_ref, out_ref, acc_ref, us_ref):
    pid = pl.program_id(0)

    @pl.when(pid == 0)
    def _init():
        for k in range(N_GPB):
            acc_ref[k] = 0.0

    x = logits_ref[...]
    t = targets_ref[0, 0, :]
    m = jnp.max(x, axis=1)
    e = jnp.exp(x - m[:, None])
    s = jnp.sum(e, axis=1)
    lse = jnp.log(s) + m
    col = jax.lax.broadcasted_iota(jnp.int32, x.shape, 1)
    tgt = jnp.sum(jnp.where(col == t[:, None], x, 0.0), axis=1)
    block_sum = jnp.sum(lse - tgt)
    g = pid // BPS
    acc_ref[g] += block_sum

    @pl.when(pid == G - 1)
    def _finish():
        c = ccb_ref[0]
        c_buf = ccb_ref[1]
        for j in range(N_GROUPS):
            us_ref[j] = u_ref[j]
        for k in range(N_GPB):
            ce_d = acc_ref[k] * INV_BPG
            gk = gid_ref[k]
            ug = u_ref[gk]
            val = ug + GAMMA * (ce_d - c - ug) + THETA * (ce_d - c - (aux_ref[k] - c_buf))
            us_ref[gk] = val
        total = 0.0
        for k in range(N_GPB):
            w = jnp.minimum(jnp.exp(us_ref[gid_ref[k]] / LAMDA), 1.0 / ALPHA)
            total = total + w * (acc_ref[k] * INV_BPG)
        out_ref[0] = total / N_GPB


@jax.jit
def _run(logits, targets3, gid, u, aux, ccb):
    return pl.pallas_call(
        _ce_kernel,
        grid=(G,),
        in_specs=[
            pl.BlockSpec(memory_space=pltpu.SMEM),
            pl.BlockSpec(memory_space=pltpu.SMEM),
            pl.BlockSpec(memory_space=pltpu.SMEM),
            pl.BlockSpec(memory_space=pltpu.SMEM),
            pl.BlockSpec((R, CLASSES), lambda i: (i, 0)),
            pl.BlockSpec((1, 1, R), lambda i: (i, 0, 0)),
        ],
        out_specs=pl.BlockSpec(memory_space=pltpu.SMEM),
        out_shape=jax.ShapeDtypeStruct((1,), jnp.float32),
        scratch_shapes=[
            pltpu.SMEM((N_GPB,), jnp.float32),
            pltpu.SMEM((N_GROUPS,), jnp.float32),
        ],
        compiler_params=pltpu.CompilerParams(
            dimension_semantics=("arbitrary",)),
    )(gid, u, aux, ccb, logits, targets3)


def kernel(epoch, logits, targets, group_ids, aux_ce_loss, u, c, c_buf):
    gid = group_ids[:: ROWS // N_GPB]
    targets3 = targets.astype(jnp.int32).reshape(G, 1, R)
    ccb = jnp.stack([jnp.asarray(c, jnp.float32), jnp.asarray(c_buf, jnp.float32)])
    out = _run(logits, targets3, gid, u, aux_ce_loss, ccb)
    return out[0]
